# baseline (speedup 1.0000x reference)
import functools

import jax
import jax.numpy as jnp
from jax import lax
from jax.experimental import pallas as pl
from jax.experimental.pallas import tpu as pltpu
from jax.experimental.pallas import tpu_sc as plsc

N_NODES = 10000
N_EDGES = 320000
D = 128

NC, NS = 2, 16
NW = NC * NS
BLK = 128

K_AGG = 80
E_PAD = NW * K_AGG * BLK
K_DEG = E_PAD // (NS * BLK)

N_PAD = 10112
ROWS_PER_TILE = N_PAD // NS

_MESH = plsc.VectorSubcoreMesh(core_axis_name="c", subcore_axis_name="s")


@functools.partial(
    pl.kernel,
    out_type=jax.ShapeDtypeStruct((NC, N_PAD, D), jnp.float32),
    mesh=_MESH,
    scratch_types=[
        pltpu.VMEM((K_DEG, BLK), jnp.int32),
        pltpu.VMEM((BLK, D), jnp.float32),
        pltpu.VMEM_SHARED((N_PAD, D), jnp.float32),
    ],
)
def _degree_kernel(edges_hbm, zc_hbm, ones_hbm, out_hbm, idx_v, ones_v, cnt_sh):
    c = lax.axis_index("c")
    s = lax.axis_index("s")
    pltpu.sync_copy(zc_hbm, cnt_sh.at[pl.ds(s * ROWS_PER_TILE, ROWS_PER_TILE)])
    pltpu.sync_copy(ones_hbm, ones_v)
    pltpu.sync_copy(edges_hbm.at[c, s], idx_v)
    plsc.subcore_barrier()

    def blk(j, carry):
        pltpu.sync_copy(ones_v, cnt_sh.at[idx_v.at[j]], add=True)
        return carry

    lax.fori_loop(0, K_DEG, blk, 0)
    plsc.subcore_barrier()
    sl = pl.ds(s * ROWS_PER_TILE, ROWS_PER_TILE)
    pltpu.sync_copy(cnt_sh.at[sl], out_hbm.at[c, sl])


BLK2 = 32
K_TILE = E_PAD // (NW * BLK2)
KC = 40
RING = 8


@functools.partial(
    pl.kernel,
    out_type=jax.ShapeDtypeStruct((NC, N_PAD, D), jnp.float32),
    mesh=_MESH,
    scratch_types=[
        pltpu.VMEM((KC, BLK2), jnp.int32),
        pltpu.VMEM((KC, BLK2), jnp.int32),
        pltpu.VMEM((RING, BLK2, D), jnp.float32),
        pltpu.VMEM_SHARED((N_PAD, D), jnp.float32),
        [pltpu.SemaphoreType.DMA] * RING,
    ],
)
def _aggregate_kernel(xws_hbm, src_hbm, dst_hbm, zr_hbm, out_hbm,
                      src_v, dst_v, rows_v, agg_sh, sems):
    c = lax.axis_index("c")
    s = lax.axis_index("s")
    pltpu.sync_copy(zr_hbm, agg_sh.at[pl.ds(s * ROWS_PER_TILE, ROWS_PER_TILE)])
    plsc.subcore_barrier()

    wid = s * NC + c
    start = wid * K_TILE

    def chunk(ci, carry):
        cbase = start + ci * KC
        pltpu.sync_copy(src_hbm.at[pl.ds(cbase, KC)], src_v)
        pltpu.sync_copy(dst_hbm.at[pl.ds(cbase, KC)], dst_v)
        for r in range(RING - 1):
            pltpu.async_copy(xws_hbm.at[src_v.at[r]], rows_v.at[r], sems[r])

        def group(i, carry2):
            j0 = RING * i
            for r in range(RING):
                j = j0 + r
                pltpu.make_async_copy(
                    xws_hbm.at[src_v.at[j]], rows_v.at[r], sems[r]).wait()
                rp = (r + RING - 1) % RING

                @pl.when(j + RING - 1 < KC)
                def _():
                    pltpu.async_copy(
                        xws_hbm.at[src_v.at[j + RING - 1]],
                        rows_v.at[rp], sems[rp])

                pltpu.sync_copy(rows_v.at[r], agg_sh.at[dst_v.at[j]],
                                add=True)
            return carry2

        lax.fori_loop(0, KC // RING, group, 0)
        return carry

    lax.fori_loop(0, K_TILE // KC, chunk, 0)

    plsc.subcore_barrier()
    sl = pl.ds(s * ROWS_PER_TILE, ROWS_PER_TILE)
    pltpu.sync_copy(agg_sh.at[sl], out_hbm.at[c, sl])


_ROWS_TC = 1000
_GRID = N_NODES // _ROWS_TC


def _norm(col):
    return lax.rsqrt(jnp.maximum(col, 1.0))


def _pre_body(dc_ref, h_ref, w_ref, o_ref):
    ns = _norm(dc_ref[0, :, 0:1])
    o_ref[...] = jnp.dot(h_ref[...] * ns, w_ref[...],
                         preferred_element_type=jnp.float32)


def _mid_body(dc_ref, agg_ref, b1_ref, w2_ref, o_ref):
    nd = _norm(dc_ref[1, :, 0:1])
    ns = _norm(dc_ref[0, :, 0:1])
    h1 = agg_ref[0] + agg_ref[1]
    h1 = h1 * nd + b1_ref[...]
    h1 = jnp.where(h1 > 0, h1, jnp.exp(jnp.minimum(h1, 0.0)) - 1.0)
    o_ref[...] = jnp.dot(h1 * ns, w2_ref[...],
                         preferred_element_type=jnp.float32)


def _post_body(dc_ref, agg_ref, b2_ref, o_ref):
    nd = _norm(dc_ref[1, :, 0:1])
    o_ref[...] = (agg_ref[0] + agg_ref[1]) * nd + b2_ref[...]


_dc_spec = pl.BlockSpec((NC, _ROWS_TC, D), lambda i: (0, i, 0))
_row_spec = pl.BlockSpec((_ROWS_TC, D), lambda i: (i, 0))
_agg_spec = pl.BlockSpec((NC, _ROWS_TC, D), lambda i: (0, i, 0))
_w_spec = pl.BlockSpec((D, D), lambda i: (0, 0))
_b_spec = pl.BlockSpec((1, D), lambda i: (0, 0))
_out_shape = jax.ShapeDtypeStruct((N_NODES, D), jnp.float32)

_pre = pl.pallas_call(
    _pre_body, grid=(_GRID,),
    in_specs=[_dc_spec, _row_spec, _w_spec],
    out_specs=_row_spec, out_shape=_out_shape)

_mid = pl.pallas_call(
    _mid_body, grid=(_GRID,),
    in_specs=[_dc_spec, _agg_spec, _b_spec, _w_spec],
    out_specs=_row_spec, out_shape=_out_shape)

_post = pl.pallas_call(
    _post_body, grid=(_GRID,),
    in_specs=[_dc_spec, _agg_spec, _b_spec],
    out_specs=_row_spec, out_shape=_out_shape)


def kernel(h, edge_index, W1, b1, W2, b2):
    src = edge_index[0]
    dst = edge_index[1]
    pad = E_PAD - N_EDGES
    junk = jnp.arange(pad, dtype=jnp.int32) % (N_PAD - N_NODES) + N_NODES
    src_g = jnp.concatenate([src, jnp.zeros((pad,), jnp.int32)])
    dst_p = jnp.concatenate([dst, junk])
    src_d = jnp.concatenate([src, junk])

    edges_deg = jnp.stack([src_d, dst_p]).reshape(NC, NS, K_DEG, BLK)
    src_g3 = src_g.reshape(NW * K_TILE, BLK2)
    dst_g3 = dst_p.reshape(NW * K_TILE, BLK2)

    ones = jnp.ones((BLK, D), jnp.float32)
    zr = jnp.zeros((ROWS_PER_TILE, D), jnp.float32)

    dcounts = _degree_kernel(edges_deg, zr, ones)

    xws1 = _pre(dcounts, h, W1)
    agg1 = _aggregate_kernel(xws1, src_g3, dst_g3, zr)
    xws2 = _mid(dcounts, agg1, b1.reshape(1, D), W2)
    agg2 = _aggregate_kernel(xws2, src_g3, dst_g3, zr)
    return _post(dcounts, agg2, b2.reshape(1, D))


# speedup vs baseline: 1.0711x; 1.0711x over previous
import functools

import jax
import jax.numpy as jnp
from jax import lax
from jax.experimental import pallas as pl
from jax.experimental.pallas import tpu as pltpu
from jax.experimental.pallas import tpu_sc as plsc

N_NODES = 10000
N_EDGES = 320000
D = 128

NC, NS = 2, 16
NW = NC * NS
BLK = 128

K_AGG = 80
E_PAD = NW * K_AGG * BLK
K_DEG = E_PAD // (NS * BLK)

N_PAD = 10112
ROWS_PER_TILE = N_PAD // NS

_MESH = plsc.VectorSubcoreMesh(core_axis_name="c", subcore_axis_name="s")


@functools.partial(
    pl.kernel,
    out_type=jax.ShapeDtypeStruct((NC, N_PAD, D), jnp.float32),
    mesh=_MESH,
    scratch_types=[
        pltpu.VMEM((K_DEG, BLK), jnp.int32),
        pltpu.VMEM((BLK, D), jnp.float32),
        pltpu.VMEM_SHARED((N_PAD, D), jnp.float32),
    ],
)
def _degree_kernel(edges_hbm, zc_hbm, ones_hbm, out_hbm, idx_v, ones_v, cnt_sh):
    c = lax.axis_index("c")
    s = lax.axis_index("s")
    pltpu.sync_copy(zc_hbm, cnt_sh.at[pl.ds(s * ROWS_PER_TILE, ROWS_PER_TILE)])
    pltpu.sync_copy(ones_hbm, ones_v)
    pltpu.sync_copy(edges_hbm.at[c, s], idx_v)
    plsc.subcore_barrier()

    def blk(j, carry):
        pltpu.sync_copy(ones_v, cnt_sh.at[idx_v.at[j]], add=True)
        return carry

    lax.fori_loop(0, K_DEG, blk, 0)
    plsc.subcore_barrier()
    sl = pl.ds(s * ROWS_PER_TILE, ROWS_PER_TILE)
    pltpu.sync_copy(cnt_sh.at[sl], out_hbm.at[c, sl])


BLK2 = 64
K_TILE = E_PAD // (NW * BLK2)
KC = 40
RING = 4


@functools.partial(
    pl.kernel,
    out_type=jax.ShapeDtypeStruct((NC, N_PAD, D), jnp.float32),
    mesh=_MESH,
    scratch_types=[
        pltpu.VMEM((KC, BLK2), jnp.int32),
        pltpu.VMEM((KC, BLK2), jnp.int32),
        pltpu.VMEM((RING, BLK2, D), jnp.float32),
        pltpu.VMEM_SHARED((N_PAD, D), jnp.float32),
        [pltpu.SemaphoreType.DMA] * RING,
    ],
)
def _aggregate_kernel(xws_hbm, src_hbm, dst_hbm, zr_hbm, out_hbm,
                      src_v, dst_v, rows_v, agg_sh, sems):
    c = lax.axis_index("c")
    s = lax.axis_index("s")
    pltpu.sync_copy(zr_hbm, agg_sh.at[pl.ds(s * ROWS_PER_TILE, ROWS_PER_TILE)])
    plsc.subcore_barrier()

    wid = s * NC + c
    start = wid * K_TILE

    def chunk(ci, carry):
        cbase = start + ci * KC
        pltpu.sync_copy(src_hbm.at[pl.ds(cbase, KC)], src_v)
        pltpu.sync_copy(dst_hbm.at[pl.ds(cbase, KC)], dst_v)
        for r in range(RING - 1):
            pltpu.async_copy(xws_hbm.at[src_v.at[r]], rows_v.at[r], sems[r])

        def group(i, carry2):
            j0 = RING * i
            for r in range(RING):
                j = j0 + r
                pltpu.make_async_copy(
                    xws_hbm.at[src_v.at[j]], rows_v.at[r], sems[r]).wait()
                rp = (r + RING - 1) % RING

                @pl.when(j + RING - 1 < KC)
                def _():
                    pltpu.async_copy(
                        xws_hbm.at[src_v.at[j + RING - 1]],
                        rows_v.at[rp], sems[rp])

                pltpu.sync_copy(rows_v.at[r], agg_sh.at[dst_v.at[j]],
                                add=True)
            return carry2

        lax.fori_loop(0, KC // RING, group, 0)
        return carry

    lax.fori_loop(0, K_TILE // KC, chunk, 0)

    plsc.subcore_barrier()
    sl = pl.ds(s * ROWS_PER_TILE, ROWS_PER_TILE)
    pltpu.sync_copy(agg_sh.at[sl], out_hbm.at[c, sl])


_ROWS_TC = 1000
_GRID = N_NODES // _ROWS_TC


def _norm(col):
    return lax.rsqrt(jnp.maximum(col, 1.0))


def _pre_body(dc_ref, h_ref, w_ref, o_ref):
    ns = _norm(dc_ref[0, :, 0:1])
    o_ref[...] = jnp.dot(h_ref[...] * ns, w_ref[...],
                         preferred_element_type=jnp.float32)


def _mid_body(dc_ref, agg_ref, b1_ref, w2_ref, o_ref):
    nd = _norm(dc_ref[1, :, 0:1])
    ns = _norm(dc_ref[0, :, 0:1])
    h1 = agg_ref[0] + agg_ref[1]
    h1 = h1 * nd + b1_ref[...]
    h1 = jnp.where(h1 > 0, h1, jnp.exp(jnp.minimum(h1, 0.0)) - 1.0)
    o_ref[...] = jnp.dot(h1 * ns, w2_ref[...],
                         preferred_element_type=jnp.float32)


def _post_body(dc_ref, agg_ref, b2_ref, o_ref):
    nd = _norm(dc_ref[1, :, 0:1])
    o_ref[...] = (agg_ref[0] + agg_ref[1]) * nd + b2_ref[...]


_dc_spec = pl.BlockSpec((NC, _ROWS_TC, D), lambda i: (0, i, 0))
_row_spec = pl.BlockSpec((_ROWS_TC, D), lambda i: (i, 0))
_agg_spec = pl.BlockSpec((NC, _ROWS_TC, D), lambda i: (0, i, 0))
_w_spec = pl.BlockSpec((D, D), lambda i: (0, 0))
_b_spec = pl.BlockSpec((1, D), lambda i: (0, 0))
_out_shape = jax.ShapeDtypeStruct((N_NODES, D), jnp.float32)

_pre = pl.pallas_call(
    _pre_body, grid=(_GRID,),
    in_specs=[_dc_spec, _row_spec, _w_spec],
    out_specs=_row_spec, out_shape=_out_shape)

_mid = pl.pallas_call(
    _mid_body, grid=(_GRID,),
    in_specs=[_dc_spec, _agg_spec, _b_spec, _w_spec],
    out_specs=_row_spec, out_shape=_out_shape)

_post = pl.pallas_call(
    _post_body, grid=(_GRID,),
    in_specs=[_dc_spec, _agg_spec, _b_spec],
    out_specs=_row_spec, out_shape=_out_shape)


def kernel(h, edge_index, W1, b1, W2, b2):
    src = edge_index[0]
    dst = edge_index[1]
    pad = E_PAD - N_EDGES
    junk = jnp.arange(pad, dtype=jnp.int32) % (N_PAD - N_NODES) + N_NODES
    src_g = jnp.concatenate([src, jnp.zeros((pad,), jnp.int32)])
    dst_p = jnp.concatenate([dst, junk])
    src_d = jnp.concatenate([src, junk])

    edges_deg = jnp.stack([src_d, dst_p]).reshape(NC, NS, K_DEG, BLK)
    src_g3 = src_g.reshape(NW * K_TILE, BLK2)
    dst_g3 = dst_p.reshape(NW * K_TILE, BLK2)

    ones = jnp.ones((BLK, D), jnp.float32)
    zr = jnp.zeros((ROWS_PER_TILE, D), jnp.float32)

    dcounts = _degree_kernel(edges_deg, zr, ones)

    xws1 = _pre(dcounts, h, W1)
    agg1 = _aggregate_kernel(xws1, src_g3, dst_g3, zr)
    xws2 = _mid(dcounts, agg1, b1.reshape(1, D), W2)
    agg2 = _aggregate_kernel(xws2, src_g3, dst_g3, zr)
    return _post(dcounts, agg2, b2.reshape(1, D))
